# Initial kernel scaffold; baseline (speedup 1.0000x reference)
import jax
import jax.numpy as jnp
from jax.experimental import pallas as pl


def kernel(card_counts, card_colors, card_shapes, card_selections, leader_rotation, follower_rotation, prev_visited_card_counts, prev_visited_card_colors, prev_visited_card_shapes, prev_visited_card_selections, nonempty_property_mask, emb_table):
    raise NotImplementedError("write your pallas kernel here")


# speedup vs baseline: 15.3622x; 15.3622x over previous
import functools

import jax
import jax.numpy as jnp
from jax.experimental import pallas as pl

B, H, W, D = 256, 25, 25, 64
HW = H * W
NUM_EMB = 60
OFFSETS = (0, 4, 12, 20, 24, 30, 36, 40, 48, 56)
GROUP_ROWS = ((0, 24), (24, 36), (36, 60))


def _embed_block(idx_ref, mask_ref, t3t_ref, out_ref):
    idx = idx_ref[0]
    mask = mask_ref[0]
    row = jax.lax.broadcasted_iota(jnp.int32, (D, HW), 0)
    acc = jnp.zeros((D, HW), jnp.float32)
    for p in range(10):
        acc = acc + jnp.where(row == idx[p][None, :], mask[p][None, :], 0.0)
    out_ref[0] = jnp.dot(t3t_ref[...], acc, preferred_element_type=jnp.float32)


@jax.jit
def kernel(card_counts, card_colors, card_shapes, card_selections,
           leader_rotation, follower_rotation,
           prev_visited_card_counts, prev_visited_card_colors,
           prev_visited_card_shapes, prev_visited_card_selections,
           nonempty_property_mask, emb_table):
    props = (card_counts, card_colors, card_shapes, card_selections,
             leader_rotation, follower_rotation,
             prev_visited_card_counts, prev_visited_card_colors,
             prev_visited_card_shapes, prev_visited_card_selections)
    idx = jnp.stack([p.reshape(B, HW) for p in props], axis=1)
    idx = idx + jnp.asarray(OFFSETS, jnp.int32)[None, :, None]
    mask = nonempty_property_mask.reshape(B, 10, HW)

    t3t = jnp.zeros((3 * D, D), jnp.float32)
    for g, (lo, hi) in enumerate(GROUP_ROWS):
        t3t = t3t.at[g * D:(g + 1) * D, lo:hi].set(emb_table[lo:hi].T)

    out = pl.pallas_call(
        _embed_block,
        grid=(B,),
        in_specs=[
            pl.BlockSpec((1, 10, HW), lambda b: (b, 0, 0)),
            pl.BlockSpec((1, 10, HW), lambda b: (b, 0, 0)),
            pl.BlockSpec((3 * D, D), lambda b: (0, 0)),
        ],
        out_specs=pl.BlockSpec((1, 3 * D, HW), lambda b: (b, 0, 0)),
        out_shape=jax.ShapeDtypeStruct((B, 3 * D, HW), jnp.float32),
    )(idx, mask, t3t)
    return out.reshape(B, 3 * D, H, W)
